# Initial kernel scaffold; baseline (speedup 1.0000x reference)
import jax
import jax.numpy as jnp
from jax.experimental import pallas as pl


def kernel(x):
    raise NotImplementedError("write your pallas kernel here")


# speedup vs baseline: 13.3525x; 13.3525x over previous
import jax
import jax.numpy as jnp
from jax import lax
from jax.experimental import pallas as pl

SIZE_IN_K = 512
SIZE_OUT_K = 128
ROWS = 100000
BLOCK_R = 2000


def _seg_sum_block(x_ref, s_ref, o_ref):
    o_ref[...] = jnp.dot(x_ref[...], s_ref[...],
                         preferred_element_type=jnp.float32)


def kernel(x):
    n, c = x.shape
    rows_i = lax.broadcasted_iota(jnp.int32, (SIZE_IN_K, SIZE_OUT_K), 0)
    cols_j = lax.broadcasted_iota(jnp.int32, (SIZE_IN_K, SIZE_OUT_K), 1)
    s = (rows_i // 4 == cols_j).astype(jnp.float32)
    grid = (n // BLOCK_R,)
    return pl.pallas_call(
        _seg_sum_block,
        grid=grid,
        in_specs=[
            pl.BlockSpec((BLOCK_R, SIZE_IN_K), lambda i: (i, 0)),
            pl.BlockSpec((SIZE_IN_K, SIZE_OUT_K), lambda i: (0, 0)),
        ],
        out_specs=pl.BlockSpec((BLOCK_R, SIZE_OUT_K), lambda i: (i, 0)),
        out_shape=jax.ShapeDtypeStruct((n, SIZE_OUT_K), jnp.float32),
    )(x, s)
